# Initial kernel scaffold; baseline (speedup 1.0000x reference)
import jax
import jax.numpy as jnp
from jax.experimental import pallas as pl


def kernel(X, W1, b1, W2, b2):
    raise NotImplementedError("write your pallas kernel here")


# speedup vs baseline: 19.1483x; 19.1483x over previous
import functools

import jax
import jax.numpy as jnp
from jax.experimental import pallas as pl
from jax.experimental.pallas import tpu as pltpu

_NNODES = 4096
_FEAT = 10
_DIM = 10
_K = 8
_ALPHA = 3.0
_T = 512


def _body(x_ref, w1_ref, b1_ref, w2_ref, b2_ref, out_ref, nv1_ref, nv2_ref):
    t = pl.program_id(1)

    @pl.when(t == 0)
    def _():
        x = x_ref[0]
        nv1_ref[...] = jnp.tanh(
            _ALPHA * (jnp.dot(x, w1_ref[...], preferred_element_type=jnp.float32)
                      + b1_ref[0][None, :]))
        nv2_ref[...] = jnp.tanh(
            _ALPHA * (jnp.dot(x, w2_ref[...], preferred_element_type=jnp.float32)
                      + b2_ref[0][None, :]))

    nv1 = nv1_ref[...]
    nv2 = nv2_ref[...]
    nv1_r = nv1_ref[pl.ds(t * _T, _T), :]
    nv2_r = nv2_ref[pl.ds(t * _T, _T), :]

    dn = (((1,), (1,)), ((), ()))
    a = (jax.lax.dot_general(nv1_r, nv2, dn, preferred_element_type=jnp.float32)
         - jax.lax.dot_general(nv2_r, nv1, dn, preferred_element_type=jnp.float32))
    adj = jnp.maximum(jnp.tanh(_ALPHA * a), 0.0)

    col = jax.lax.broadcasted_iota(jnp.int32, (_T, _NNODES), 1)
    work = adj
    sel_mask = jnp.zeros((_T, _NNODES), dtype=jnp.bool_)
    for _ in range(_K):
        m = jnp.max(work, axis=1, keepdims=True)
        idxs = jnp.where(work == m, col, _NNODES)
        first = jnp.min(idxs, axis=1, keepdims=True)
        sel = col == first
        sel_mask = jnp.logical_or(sel_mask, sel)
        work = jnp.where(sel, -1.0, work)

    out_ref[0] = jnp.where(sel_mask, adj, 0.0)


@jax.jit
def kernel(X, W1, b1, W2, b2):
    B, N, F = X.shape
    grid = (B, N // _T)
    return pl.pallas_call(
        _body,
        grid=grid,
        in_specs=[
            pl.BlockSpec((1, N, F), lambda b, t: (b, 0, 0)),
            pl.BlockSpec((F, _DIM), lambda b, t: (0, 0)),
            pl.BlockSpec((1, _DIM), lambda b, t: (0, 0)),
            pl.BlockSpec((F, _DIM), lambda b, t: (0, 0)),
            pl.BlockSpec((1, _DIM), lambda b, t: (0, 0)),
        ],
        out_specs=pl.BlockSpec((1, _T, N), lambda b, t: (b, t, 0)),
        out_shape=jax.ShapeDtypeStruct((B, N, N), jnp.float32),
        scratch_shapes=[
            pltpu.VMEM((N, _DIM), jnp.float32),
            pltpu.VMEM((N, _DIM), jnp.float32),
        ],
    )(X, W1, b1.reshape(1, -1), W2, b2.reshape(1, -1))


# speedup vs baseline: 25.3931x; 1.3261x over previous
import functools

import jax
import jax.numpy as jnp
from jax.experimental import pallas as pl
from jax.experimental.pallas import tpu as pltpu

_NNODES = 4096
_FEAT = 10
_DIM = 10
_K = 8
_ALPHA = 3.0
_T = 512


def _body(x_ref, w1_ref, b1_ref, w2_ref, b2_ref, out_ref, nv1_ref, nv2_ref):
    t = pl.program_id(1)

    @pl.when(t == 0)
    def _():
        x = x_ref[0]
        nv1_ref[...] = jnp.tanh(
            _ALPHA * (jnp.dot(x, w1_ref[...], preferred_element_type=jnp.float32)
                      + b1_ref[0][None, :]))
        nv2_ref[...] = jnp.tanh(
            _ALPHA * (jnp.dot(x, w2_ref[...], preferred_element_type=jnp.float32)
                      + b2_ref[0][None, :]))

    nv1 = nv1_ref[...]
    nv2 = nv2_ref[...]
    nv1_r = nv1_ref[pl.ds(t * _T, _T), :]
    nv2_r = nv2_ref[pl.ds(t * _T, _T), :]

    dn = (((1,), (1,)), ((), ()))
    a = (jax.lax.dot_general(nv1_r, nv2, dn, preferred_element_type=jnp.float32)
         - jax.lax.dot_general(nv2_r, nv1, dn, preferred_element_type=jnp.float32))
    adj = jnp.maximum(jnp.tanh(_ALPHA * a), 0.0)

    col = jax.lax.broadcasted_iota(jnp.int32, (_T, _NNODES), 1)
    work = adj
    for _ in range(_K):
        m = jnp.max(work, axis=1, keepdims=True)
        idxs = jnp.where(work == m, col, _NNODES)
        first = jnp.min(idxs, axis=1, keepdims=True)
        sel = col == first
        work = jnp.where(sel, -1.0, work)

    out_ref[0] = jnp.where(work < 0.0, adj, 0.0)


@jax.jit
def kernel(X, W1, b1, W2, b2):
    B, N, F = X.shape
    grid = (B, N // _T)
    return pl.pallas_call(
        _body,
        grid=grid,
        in_specs=[
            pl.BlockSpec((1, N, F), lambda b, t: (b, 0, 0)),
            pl.BlockSpec((F, _DIM), lambda b, t: (0, 0)),
            pl.BlockSpec((1, _DIM), lambda b, t: (0, 0)),
            pl.BlockSpec((F, _DIM), lambda b, t: (0, 0)),
            pl.BlockSpec((1, _DIM), lambda b, t: (0, 0)),
        ],
        out_specs=pl.BlockSpec((1, _T, N), lambda b, t: (b, t, 0)),
        out_shape=jax.ShapeDtypeStruct((B, N, N), jnp.float32),
        scratch_shapes=[
            pltpu.VMEM((N, _DIM), jnp.float32),
            pltpu.VMEM((N, _DIM), jnp.float32),
        ],
    )(X, W1, b1.reshape(1, -1), W2, b2.reshape(1, -1))


# speedup vs baseline: 29.4442x; 1.1595x over previous
import functools

import jax
import jax.numpy as jnp
from jax.experimental import pallas as pl
from jax.experimental.pallas import tpu as pltpu

_NNODES = 4096
_FEAT = 10
_DIM = 10
_K = 8
_ALPHA = 3.0
_T = 512


def _body(x_ref, w1_ref, b1_ref, w2_ref, b2_ref, out_ref, nv1_ref, nv2_ref):
    t = pl.program_id(1)

    @pl.when(t == 0)
    def _():
        x = x_ref[0]
        nv1_ref[...] = jnp.tanh(
            _ALPHA * (jnp.dot(x, w1_ref[...], preferred_element_type=jnp.float32)
                      + b1_ref[0][None, :]))
        nv2_ref[...] = jnp.tanh(
            _ALPHA * (jnp.dot(x, w2_ref[...], preferred_element_type=jnp.float32)
                      + b2_ref[0][None, :]))

    nv1 = nv1_ref[...]
    nv2 = nv2_ref[...]
    nv1_r = nv1_ref[pl.ds(t * _T, _T), :]
    nv2_r = nv2_ref[pl.ds(t * _T, _T), :]

    dn = (((1,), (1,)), ((), ()))
    a = (jax.lax.dot_general(nv1_r, nv2, dn, preferred_element_type=jnp.float32)
         - jax.lax.dot_general(nv2_r, nv1, dn, preferred_element_type=jnp.float32))
    adj = jnp.maximum(jnp.tanh(_ALPHA * a), 0.0)

    col = jax.lax.broadcasted_iota(
        jnp.int32, (_T, _NNODES), 1).astype(jnp.float32)
    work = adj
    for _ in range(_K):
        m = jnp.max(work, axis=1, keepdims=True)
        idxs = jnp.where(work == m, col, 8192.0)
        first = jnp.min(idxs, axis=1, keepdims=True)
        work = jnp.where(col == first, -1.0, work)

    out_ref[0] = jnp.where(work < 0.0, adj, 0.0)


@jax.jit
def kernel(X, W1, b1, W2, b2):
    B, N, F = X.shape
    grid = (B, N // _T)
    return pl.pallas_call(
        _body,
        grid=grid,
        in_specs=[
            pl.BlockSpec((1, N, F), lambda b, t: (b, 0, 0)),
            pl.BlockSpec((F, _DIM), lambda b, t: (0, 0)),
            pl.BlockSpec((1, _DIM), lambda b, t: (0, 0)),
            pl.BlockSpec((F, _DIM), lambda b, t: (0, 0)),
            pl.BlockSpec((1, _DIM), lambda b, t: (0, 0)),
        ],
        out_specs=pl.BlockSpec((1, _T, N), lambda b, t: (b, t, 0)),
        out_shape=jax.ShapeDtypeStruct((B, N, N), jnp.float32),
        scratch_shapes=[
            pltpu.VMEM((N, _DIM), jnp.float32),
            pltpu.VMEM((N, _DIM), jnp.float32),
        ],
    )(X, W1, b1.reshape(1, -1), W2, b2.reshape(1, -1))
